# baseline (speedup 1.0000x reference)
import jax
import jax.numpy as jnp
from jax import lax
from jax.experimental import pallas as pl
from jax.experimental.pallas import tpu as pltpu
from jax.experimental.pallas import tpu_sc as plsc

DIM = 256
NE = 8192
ROWS = 16 * 576
RB = 256
NB = ROWS // RB

_W = 1664
_WINDOWS = [(b, min(b + _W, NE)) for b in range(0, NE, _W)]

_NC = 2
_NS = 16
_NW = _NC * _NS
_BPW = ROWS // _NW


def _dist_body(x_ref, e_ref, rn_ref, en_ref, ind_ref, dsum_ref, acc_ref):
    i = pl.program_id(0)

    @pl.when(i == 0)
    def _init():
        acc_ref[0] = 0.0

    x = x_ref[...]
    xb = x.astype(jnp.bfloat16)
    eb = e_ref[...].astype(jnp.bfloat16)
    s = lax.dot_general(xb, eb, (((1,), (0,)), ((), ())),
                        preferred_element_type=jnp.float32)
    v = -((rn_ref[...] - 2.0 * s) + en_ref[...])

    accv = jnp.full((RB, 1), -jnp.inf, jnp.float32)
    acci = jnp.zeros((RB, 1), jnp.int32)
    selv = jnp.full((RB, 1), -jnp.inf, jnp.float32)
    for b0, b1 in _WINDOWS:
        w = v[:, b0:b1]
        wv = jnp.max(w, axis=1, keepdims=True)
        iota = lax.broadcasted_iota(jnp.int32, (RB, b1 - b0), 1) + b0
        wi = jnp.min(jnp.where(w == wv, iota, NE), axis=1, keepdims=True)
        upd = wv > accv
        accv = jnp.where(upd, wv.astype(jnp.bfloat16).astype(jnp.float32), accv)
        acci = jnp.where(upd, wi, acci)
        selv = jnp.where(upd, wv, selv)

    ind_ref[...] = acci
    acc_ref[0] += jnp.sum(-selv)

    @pl.when(i == NB - 1)
    def _fin():
        dsum_ref[0, 0] = acc_ref[0] / (ROWS * DIM)


_dist_call = pl.pallas_call(
    _dist_body,
    grid=(NB,),
    in_specs=[
        pl.BlockSpec((RB, DIM), lambda i: (i, 0)),
        pl.BlockSpec((DIM, NE), lambda i: (0, 0)),
        pl.BlockSpec((RB, 1), lambda i: (i, 0)),
        pl.BlockSpec((1, NE), lambda i: (0, 0)),
    ],
    out_specs=[
        pl.BlockSpec((RB, 1), lambda i: (i, 0)),
        pl.BlockSpec(memory_space=pltpu.SMEM),
    ],
    out_shape=[
        jax.ShapeDtypeStruct((ROWS, 1), jnp.int32),
        jax.ShapeDtypeStruct((1, 1), jnp.float32),
    ],
    scratch_shapes=[
        pltpu.SMEM((1,), jnp.float32),
    ],
)


def _gather_body(table_hbm, idx_hbm, out_hbm, idx_v, rows_v, sem):
    wid = lax.axis_index("s") * _NC + lax.axis_index("c")
    base = wid * _BPW
    pltpu.sync_copy(idx_hbm.at[pl.ds(base, _BPW)], idx_v)
    pltpu.async_copy(table_hbm.at[idx_v], rows_v, sem).wait()
    pltpu.sync_copy(rows_v, out_hbm.at[pl.ds(base, _BPW)])


_gather_call_cache = []


def _gather_call():
    if not _gather_call_cache:
        _gather_call_cache.append(pl.kernel(
            _gather_body,
            out_type=jax.ShapeDtypeStruct((ROWS, DIM), jnp.float32),
            mesh=plsc.VectorSubcoreMesh(core_axis_name="c", subcore_axis_name="s"),
            scratch_types=[
                pltpu.VMEM((_BPW,), jnp.int32),
                pltpu.VMEM((_BPW, DIM), jnp.float32),
                pltpu.SemaphoreType.DMA,
            ],
        ))
    return _gather_call_cache[0]


def kernel(input, embed):
    flat = input.reshape(ROWS, DIM)
    rn = (flat**2).sum(axis=1, keepdims=True)
    en = (embed**2).sum(axis=0, keepdims=True)
    inds2d, dsum = _dist_call(flat, embed, rn, en)
    table = embed.T
    q = _gather_call()(table, inds2d.reshape(ROWS))
    quantize = q.reshape(input.shape)
    diff = dsum.reshape(())
    embed_ind = inds2d.reshape(input.shape[:-1])
    return quantize, diff, embed_ind
